# baseline (speedup 1.0000x reference)
import functools

import jax
import jax.numpy as jnp
from jax.experimental import pallas as pl


def _linear_kernel(x_ref, w_ref, b_ref, o_ref, *, relu):
    acc = jnp.dot(x_ref[...], w_ref[...], preferred_element_type=jnp.float32)
    acc = acc + b_ref[...]
    if relu:
        acc = jnp.maximum(acc, 0.0)
    o_ref[...] = acc


def _linear(x, W, b, relu, bn=512):
    M, K = x.shape
    _, N = W.shape
    bn = min(bn, N)
    while N % bn:
        bn -= 128
    grid = (N // bn,)
    return pl.pallas_call(
        functools.partial(_linear_kernel, relu=relu),
        grid=grid,
        in_specs=[
            pl.BlockSpec((M, K), lambda j: (0, 0)),
            pl.BlockSpec((K, bn), lambda j: (0, j)),
            pl.BlockSpec((1, bn), lambda j: (0, j)),
        ],
        out_specs=pl.BlockSpec((M, bn), lambda j: (0, j)),
        out_shape=jax.ShapeDtypeStruct((M, N), jnp.float32),
    )(x, W, b.reshape(1, N))


def _rq_body(r, cb_ref, n_layers):
    zq = jnp.zeros_like(r)
    lsum = jnp.float32(0.0)
    idx_rows = []
    for i in range(n_layers):
        cb = cb_ref[i]
        d = (jnp.sum(r * r, axis=1, keepdims=True)
             - 2.0 * jax.lax.dot_general(
                 r, cb, (((1,), (1,)), ((), ())),
                 preferred_element_type=jnp.float32)
             + jnp.sum(cb * cb, axis=1)[None, :])
        idx = jnp.argmin(d, axis=1)
        onehot = (idx[:, None] == jax.lax.broadcasted_iota(
            jnp.int32, d.shape, 1)).astype(jnp.float32)
        q = jnp.dot(onehot, cb, preferred_element_type=jnp.float32,
                    precision=jax.lax.Precision.HIGHEST)
        lsum = lsum + jnp.sum((q - r) ** 2)
        q_st = r + (q - r)
        zq = zq + q_st
        r = r - q_st
        idx_rows.append(idx)
    return zq, jnp.stack(idx_rows, axis=0), lsum


_SPLIT = 4


def _mid_kernel(h_ref, w2_ref, b2_ref, w3_ref, b3_ref, w4_ref, b4_ref,
                cb_ref, dw1_ref, db1_ref, dw2_ref, db2_ref, dw3_ref, db3_ref,
                out_ref, zq_ref, idx_ref, loss_ref, *, n_layers):
    B = h_ref.shape[0]
    m = B // _SPLIT
    lsums = []
    for s in range(_SPLIT):
        sl = pl.ds(s * m, m)
        h = h_ref[sl, :]
        h = jnp.maximum(jnp.dot(h, w2_ref[...],
                                preferred_element_type=jnp.float32)
                        + b2_ref[...], 0.0)
        h = jnp.maximum(jnp.dot(h, w3_ref[...],
                                preferred_element_type=jnp.float32)
                        + b3_ref[...], 0.0)
        e = jnp.dot(h, w4_ref[...],
                    preferred_element_type=jnp.float32) + b4_ref[...]
        zq, idx, lsum = _rq_body(e, cb_ref, n_layers)
        zq_ref[sl, :] = zq
        idx_ref[:, sl] = idx
        lsums.append(lsum)
        g = jnp.maximum(jnp.dot(zq, dw1_ref[...],
                                preferred_element_type=jnp.float32)
                        + db1_ref[...], 0.0)
        g = jnp.maximum(jnp.dot(g, dw2_ref[...],
                                preferred_element_type=jnp.float32)
                        + db2_ref[...], 0.0)
        g = jnp.maximum(jnp.dot(g, dw3_ref[...],
                                preferred_element_type=jnp.float32)
                        + db3_ref[...], 0.0)
        out_ref[sl, :] = g
    total = lsums[0]
    for x in lsums[1:]:
        total = total + x
    loss_ref[...] = jnp.reshape(total, (1, 1))


def _mid(h1, enc, cb, dec):
    B = h1.shape[0]
    L = cb.shape[0]
    E = cb.shape[2]
    (W2, b2), (W3, b3), (W4, b4) = enc[1], enc[2], enc[3]
    (dW1, db1), (dW2, db2), (dW3, db3) = dec[0], dec[1], dec[2]
    N_out = dW3.shape[1]
    out, zq, idxT, lsum = pl.pallas_call(
        functools.partial(_mid_kernel, n_layers=L),
        out_shape=(
            jax.ShapeDtypeStruct((B, N_out), jnp.float32),
            jax.ShapeDtypeStruct((B, E), jnp.float32),
            jax.ShapeDtypeStruct((L, B), jnp.int32),
            jax.ShapeDtypeStruct((1, 1), jnp.float32),
        ),
    )(h1, W2, b2.reshape(1, -1), W3, b3.reshape(1, -1), W4, b4.reshape(1, -1),
      cb, dW1, db1.reshape(1, -1), dW2, db2.reshape(1, -1),
      dW3, db3.reshape(1, -1))
    loss = 1.25 * lsum[0, 0] / (B * E)
    return out, zq, loss, idxT.T


def _modality(x, enc, dec, cb):
    W1, b1 = enc[0]
    h1 = _linear(x, W1, b1, relu=True)
    h3, zq, loss, indices = _mid(h1, enc, cb, dec)
    dW4, db4 = dec[3]
    out = _linear(h3, dW4, db4, relu=False)
    return out, loss, indices, zq


def kernel(text_x, image_x, t_enc, t_dec, i_enc, i_dec, t_cb, i_cb):
    text_out, text_rq_loss, text_indices, z_q_text = _modality(
        text_x, t_enc, t_dec, t_cb)
    image_out, image_rq_loss, image_indices, z_q_image = _modality(
        image_x, i_enc, i_dec, i_cb)
    return (text_out, image_out, text_rq_loss, image_rq_loss,
            text_indices, image_indices, z_q_text, z_q_image)


# speedup vs baseline: 1.1434x; 1.1434x over previous
import jax
import jax.numpy as jnp
from jax.experimental import pallas as pl
from jax.experimental.pallas import tpu as pltpu

_NB = 512


def _rq_body(r, cb_ref, n_layers):
    zq = jnp.zeros_like(r)
    lsum = jnp.float32(0.0)
    idx_rows = []
    for i in range(n_layers):
        cb = cb_ref[i]
        d = (jnp.sum(r * r, axis=1, keepdims=True)
             - 2.0 * jax.lax.dot_general(
                 r, cb, (((1,), (1,)), ((), ())),
                 preferred_element_type=jnp.float32)
             + jnp.sum(cb * cb, axis=1)[None, :])
        idx = jnp.argmin(d, axis=1)
        onehot = (idx[:, None] == jax.lax.broadcasted_iota(
            jnp.int32, d.shape, 1)).astype(jnp.float32)
        q = jnp.dot(onehot, cb, preferred_element_type=jnp.float32,
                    precision=jax.lax.Precision.HIGHEST)
        lsum = lsum + jnp.sum((q - r) ** 2)
        q_st = r + (q - r)
        zq = zq + q_st
        r = r - q_st
        idx_rows.append(idx)
    return zq, jnp.stack(idx_rows, axis=0), lsum


def _ix(ix):
    return ix[0] if isinstance(ix, (tuple, list)) else ix


def _mega_kernel(xt_hbm, xi_hbm,
                 w1t_hbm, w2t_hbm, w3t_hbm, w4t_ref,
                 dw1t_ref, dw2t_hbm, dw3t_hbm, dw4t_hbm,
                 w1i_hbm, w2i_hbm, w3i_hbm, w4i_ref,
                 dw1i_ref, dw2i_hbm, dw3i_hbm, dw4i_hbm,
                 b1t_ref, b2t_ref, b3t_ref, b4t_ref,
                 db1t_ref, db2t_ref, db3t_ref, db4t_ref,
                 b1i_ref, b2i_ref, b3i_ref, b4i_ref,
                 db1i_ref, db2i_ref, db3i_ref, db4i_ref,
                 cbt_ref, cbi_ref,
                 tout_hbm, iout_hbm, tloss_ref, iloss_ref,
                 tidx_ref, iidx_ref, tzq_ref, izq_ref,
                 xbuf, abuf, bbuf, w3bt, dw2bt, msem):
    n_cb = cbt_ref.shape[0]

    xt_cp = pltpu.make_async_copy(xt_hbm, xbuf.at[:, pl.ds(0, 4096)],
                                  msem.at[0])
    xt_cp.start()
    w3t_cp = pltpu.make_async_copy(w3t_hbm, w3bt, msem.at[1])
    w3t_cp.start()
    dw2t_cp = pltpu.make_async_copy(dw2t_hbm, dw2bt, msem.at[2])
    dw2t_cp.start()
    w3i_cp = pltpu.make_async_copy(w3i_hbm, w3bt, msem.at[3])
    dw2i_cp = pltpu.make_async_copy(dw2i_hbm, dw2bt, msem.at[4])
    xi_cp = pltpu.make_async_copy(xi_hbm, xbuf.at[:, pl.ds(2048, 768)],
                                  msem.at[5])

    def modality(x_col0, x_width, w1_hbm, w2_hbm, dw3_hbm, dw4_hbm,
                 w3b, dw2b, w3_cp, dw2_cp, enc_refs, dec_refs, cb_ref,
                 out_hbm, out_cols, loss_ref, idx_ref, zq_ref, after_l2,
                 after_d3):
        (b1, b2, b3, w4, b4) = enc_refs
        (dw1, db1, db2, db3, db4) = dec_refs
        k1 = w1_hbm.shape[0]

        def l1_body(ix, w_ref):
            j = _ix(ix)
            h = jnp.dot(xbuf[:, pl.ds(x_col0, x_width)], w_ref[...],
                        preferred_element_type=jnp.float32)
            abuf[:, pl.ds(j * _NB, _NB)] = jnp.maximum(
                h + b1[:, pl.ds(j * _NB, _NB)], 0.0)

        pltpu.emit_pipeline(
            l1_body, grid=(2048 // _NB,),
            in_specs=[pl.BlockSpec((k1, _NB), lambda j: (0, j))],
            _explicit_indices=True,
        )(w1_hbm)

        def l2_body(ix, w_ref):
            j = _ix(ix)
            h = jnp.dot(abuf[...], w_ref[...],
                        preferred_element_type=jnp.float32)
            bbuf[:, pl.ds(j * _NB, _NB)] = jnp.maximum(
                h + b2[:, pl.ds(j * _NB, _NB)], 0.0)

        pltpu.emit_pipeline(
            l2_body, grid=(1024 // _NB,),
            in_specs=[pl.BlockSpec((2048, _NB), lambda j: (0, j))],
            _explicit_indices=True,
        )(w2_hbm)

        after_l2()

        w3_cp.wait()
        h3 = jnp.maximum(jnp.dot(bbuf[...], w3b[...],
                                 preferred_element_type=jnp.float32)
                         + b3[:, :], 0.0)
        e = jnp.dot(h3, w4[:, :],
                    preferred_element_type=jnp.float32) + b4[:, :]
        zq, idx, lsum = _rq_body(e, cb_ref, n_cb)
        zq_ref[:, :] = zq
        idx_ref[:, :] = idx
        loss_ref[...] = jnp.reshape(lsum, (1, 1))
        dw2_cp.wait()
        g1 = jnp.maximum(jnp.dot(zq, dw1[:, :],
                                 preferred_element_type=jnp.float32)
                         + db1[:, :], 0.0)
        g2 = jnp.maximum(jnp.dot(g1, dw2b[...],
                                 preferred_element_type=jnp.float32)
                         + db2[:, :], 0.0)
        bbuf[:, :] = g2

        nb3 = 256

        def d3_body(ix, w_ref):
            j = _ix(ix)
            g = jnp.dot(bbuf[...], w_ref[...],
                        preferred_element_type=jnp.float32)
            xbuf[:, pl.ds(j * nb3, nb3)] = jnp.maximum(
                g + db3[:, pl.ds(j * nb3, nb3)], 0.0)

        pltpu.emit_pipeline(
            d3_body, grid=(2048 // nb3,),
            in_specs=[pl.BlockSpec((1024, nb3), lambda j: (0, j))],
            _explicit_indices=True,
        )(dw3_hbm)

        after_d3()

        nb4 = 384 if out_cols % _NB else _NB

        def d4_body(ix, w_ref, o_ref):
            j = _ix(ix)
            o = jnp.dot(xbuf[:, pl.ds(0, 2048)], w_ref[...],
                        preferred_element_type=jnp.float32)
            o_ref[...] = o + db4[:, pl.ds(j * nb4, nb4)]

        pltpu.emit_pipeline(
            d4_body, grid=(out_cols // nb4,),
            in_specs=[pl.BlockSpec((2048, nb4), lambda j: (0, j))],
            out_specs=[pl.BlockSpec((1024, nb4), lambda j: (0, j))],
            _explicit_indices=True,
        )(dw4_hbm, out_hbm)

    xt_cp.wait()
    modality(0, 4096, w1t_hbm, w2t_hbm, dw3t_hbm, dw4t_hbm, w3bt, dw2bt,
             w3t_cp, dw2t_cp,
             (b1t_ref, b2t_ref, b3t_ref, w4t_ref, b4t_ref),
             (dw1t_ref, db1t_ref, db2t_ref, db3t_ref, db4t_ref),
             cbt_ref, tout_hbm, 4096, tloss_ref, tidx_ref, tzq_ref,
             lambda: xi_cp.start(),
             lambda: (w3i_cp.start(), dw2i_cp.start()))

    xi_cp.wait()
    modality(2048, 768, w1i_hbm, w2i_hbm, dw3i_hbm, dw4i_hbm, w3bt, dw2bt,
             w3i_cp, dw2i_cp,
             (b1i_ref, b2i_ref, b3i_ref, w4i_ref, b4i_ref),
             (dw1i_ref, db1i_ref, db2i_ref, db3i_ref, db4i_ref),
             cbi_ref, iout_hbm, 768, iloss_ref, iidx_ref, izq_ref,
             lambda: None, lambda: None)


def _mega(text_x, image_x, t_enc, t_dec, i_enc, i_dec, t_cb, i_cb):
    B = text_x.shape[0]
    L, K, E = t_cb.shape
    f32 = jnp.float32
    any_spec = pl.BlockSpec(memory_space=pl.ANY)
    vmem_spec = pl.BlockSpec(memory_space=pltpu.MemorySpace.VMEM)
    (W1t, b1t), (W2t, b2t), (W3t, b3t), (W4t, b4t) = t_enc
    (dW1t, db1t), (dW2t, db2t), (dW3t, db3t), (dW4t, db4t) = t_dec
    (W1i, b1i), (W2i, b2i), (W3i, b3i), (W4i, b4i) = i_enc
    (dW1i, db1i), (dW2i, db2i), (dW3i, db3i), (dW4i, db4i) = i_dec
    r2 = lambda b: b.reshape(1, -1)
    outs = pl.pallas_call(
        _mega_kernel,
        in_specs=(
            [any_spec, any_spec]
            + [any_spec, any_spec, any_spec, vmem_spec,
               vmem_spec, any_spec, any_spec, any_spec]
            + [any_spec, any_spec, any_spec, vmem_spec,
               vmem_spec, any_spec, any_spec, any_spec]
            + [vmem_spec] * 16
            + [vmem_spec, vmem_spec]
        ),
        out_specs=(any_spec, any_spec, vmem_spec, vmem_spec,
                   vmem_spec, vmem_spec, vmem_spec, vmem_spec),
        out_shape=(
            jax.ShapeDtypeStruct((B, 4096), f32),
            jax.ShapeDtypeStruct((B, 768), f32),
            jax.ShapeDtypeStruct((1, 1), f32),
            jax.ShapeDtypeStruct((1, 1), f32),
            jax.ShapeDtypeStruct((L, B), jnp.int32),
            jax.ShapeDtypeStruct((L, B), jnp.int32),
            jax.ShapeDtypeStruct((B, E), f32),
            jax.ShapeDtypeStruct((B, E), f32),
        ),
        scratch_shapes=[
            pltpu.VMEM((1024, 4096), f32),
            pltpu.VMEM((1024, 2048), f32),
            pltpu.VMEM((1024, 1024), f32),
            pltpu.VMEM((1024, 512), f32),
            pltpu.VMEM((512, 1024), f32),
            pltpu.SemaphoreType.DMA((8,)),
        ],
    )(text_x, image_x,
      W1t, W2t, W3t, W4t, dW1t, dW2t, dW3t, dW4t,
      W1i, W2i, W3i, W4i, dW1i, dW2i, dW3i, dW4i,
      r2(b1t), r2(b2t), r2(b3t), r2(b4t),
      r2(db1t), r2(db2t), r2(db3t), r2(db4t),
      r2(b1i), r2(b2i), r2(b3i), r2(b4i),
      r2(db1i), r2(db2i), r2(db3i), r2(db4i),
      t_cb, i_cb)
    (tout, iout, tls, ils, tidxT, iidxT, tzq, izq) = outs
    t_loss = 1.25 * tls[0, 0] / (B * E)
    i_loss = 1.25 * ils[0, 0] / (B * E)
    return (tout, iout, t_loss, i_loss, tidxT.T, iidxT.T, tzq, izq)


def kernel(text_x, image_x, t_enc, t_dec, i_enc, i_dec, t_cb, i_cb):
    return _mega(text_x, image_x, t_enc, t_dec, i_enc, i_dec, t_cb, i_cb)


# speedup vs baseline: 1.2469x; 1.0905x over previous
import jax
import jax.numpy as jnp
from jax.experimental import pallas as pl
from jax.experimental.pallas import tpu as pltpu

_NB = 512


def _rq_body(r, cb_ref, n_layers):
    zq = jnp.zeros_like(r)
    lsum = jnp.float32(0.0)
    idx_rows = []
    for i in range(n_layers):
        cb = cb_ref[i]
        d = (jnp.sum(r * r, axis=1, keepdims=True)
             - 2.0 * jax.lax.dot_general(
                 r, cb, (((1,), (1,)), ((), ())),
                 preferred_element_type=jnp.float32)
             + jnp.sum(cb * cb, axis=1)[None, :])
        idx = jnp.argmin(d, axis=1)
        onehot = (idx[:, None] == jax.lax.broadcasted_iota(
            jnp.int32, d.shape, 1)).astype(jnp.float32)
        cb0 = cb.astype(jnp.bfloat16).astype(jnp.float32)
        r1 = cb - cb0
        cb1 = r1.astype(jnp.bfloat16).astype(jnp.float32)
        cb2 = r1 - cb1
        q = (jnp.dot(onehot, cb0, preferred_element_type=jnp.float32)
             + jnp.dot(onehot, cb1, preferred_element_type=jnp.float32)
             + jnp.dot(onehot, cb2, preferred_element_type=jnp.float32))
        lsum = lsum + jnp.sum((q - r) ** 2)
        q_st = r + (q - r)
        zq = zq + q_st
        r = r - q_st
        idx_rows.append(idx)
    return zq, jnp.stack(idx_rows, axis=0), lsum


def _ix(ix):
    return ix[0] if isinstance(ix, (tuple, list)) else ix


def _mega_kernel(xt_hbm, xi_hbm,
                 w1t_hbm, w2t_hbm, w3t_hbm, w4t_ref,
                 dw1t_ref, dw2t_hbm, dw3t_hbm, dw4t_hbm,
                 w1i_hbm, w2i_hbm, w3i_hbm, w4i_ref,
                 dw1i_ref, dw2i_hbm, dw3i_hbm, dw4i_hbm,
                 b1t_ref, b2t_ref, b3t_ref, b4t_ref,
                 db1t_ref, db2t_ref, db3t_ref, db4t_ref,
                 b1i_ref, b2i_ref, b3i_ref, b4i_ref,
                 db1i_ref, db2i_ref, db3i_ref, db4i_ref,
                 cbt_ref, cbi_ref,
                 tout_hbm, iout_hbm, tloss_ref, iloss_ref,
                 tidx_ref, iidx_ref, tzq_ref, izq_ref,
                 xbuf, abuf, bbuf, w3bt, dw2bt, msem):
    n_cb = cbt_ref.shape[0]

    xt_cp = pltpu.make_async_copy(xt_hbm, xbuf.at[:, pl.ds(0, 4096)],
                                  msem.at[0])
    xt_cp.start()
    w3t_cp = pltpu.make_async_copy(w3t_hbm, w3bt, msem.at[1])
    w3t_cp.start()
    dw2t_cp = pltpu.make_async_copy(dw2t_hbm, dw2bt, msem.at[2])
    dw2t_cp.start()
    w3i_cp = pltpu.make_async_copy(w3i_hbm, w3bt, msem.at[3])
    dw2i_cp = pltpu.make_async_copy(dw2i_hbm, dw2bt, msem.at[4])
    xi_cp = pltpu.make_async_copy(xi_hbm, xbuf.at[:, pl.ds(2048, 768)],
                                  msem.at[5])

    def modality(x_col0, x_width, w1_hbm, w2_hbm, dw3_hbm, dw4_hbm,
                 w3b, dw2b, w3_cp, dw2_cp, enc_refs, dec_refs, cb_ref,
                 out_hbm, out_cols, loss_ref, idx_ref, zq_ref, after_l2,
                 after_d3):
        (b1, b2, b3, w4, b4) = enc_refs
        (dw1, db1, db2, db3, db4) = dec_refs
        k1 = w1_hbm.shape[0]

        def l1_body(ix, w_ref):
            j = _ix(ix)
            h = jnp.dot(xbuf[:, pl.ds(x_col0, x_width)], w_ref[...],
                        preferred_element_type=jnp.float32)
            abuf[:, pl.ds(j * _NB, _NB)] = jnp.maximum(
                h + b1[:, pl.ds(j * _NB, _NB)], 0.0)

        pltpu.emit_pipeline(
            l1_body, grid=(2048 // _NB,),
            in_specs=[pl.BlockSpec((k1, _NB), lambda j: (0, j))],
            _explicit_indices=True,
        )(w1_hbm)

        def l2_body(ix, w_ref):
            j = _ix(ix)
            h = jnp.dot(abuf[...], w_ref[...],
                        preferred_element_type=jnp.float32)
            bbuf[:, pl.ds(j * _NB, _NB)] = jnp.maximum(
                h + b2[:, pl.ds(j * _NB, _NB)], 0.0)

        pltpu.emit_pipeline(
            l2_body, grid=(1024 // _NB,),
            in_specs=[pl.BlockSpec((2048, _NB), lambda j: (0, j))],
            _explicit_indices=True,
        )(w2_hbm)

        after_l2()

        w3_cp.wait()
        h3 = jnp.maximum(jnp.dot(bbuf[...], w3b[...],
                                 preferred_element_type=jnp.float32)
                         + b3[:, :], 0.0)
        e = jnp.dot(h3, w4[:, :],
                    preferred_element_type=jnp.float32) + b4[:, :]
        zq, idx, lsum = _rq_body(e, cb_ref, n_cb)
        zq_ref[:, :] = zq
        idx_ref[:, :] = idx
        loss_ref[...] = jnp.reshape(lsum, (1, 1))
        dw2_cp.wait()
        g1 = jnp.maximum(jnp.dot(zq, dw1[:, :],
                                 preferred_element_type=jnp.float32)
                         + db1[:, :], 0.0)
        g2 = jnp.maximum(jnp.dot(g1, dw2b[...],
                                 preferred_element_type=jnp.float32)
                         + db2[:, :], 0.0)
        bbuf[:, :] = g2

        nb3 = 256

        def d3_body(ix, w_ref):
            j = _ix(ix)
            g = jnp.dot(bbuf[...], w_ref[...],
                        preferred_element_type=jnp.float32)
            xbuf[:, pl.ds(j * nb3, nb3)] = jnp.maximum(
                g + db3[:, pl.ds(j * nb3, nb3)], 0.0)

        pltpu.emit_pipeline(
            d3_body, grid=(2048 // nb3,),
            in_specs=[pl.BlockSpec((1024, nb3), lambda j: (0, j))],
            _explicit_indices=True,
        )(dw3_hbm)

        after_d3()

        nb4 = 384 if out_cols % _NB else _NB

        def d4_body(ix, w_ref, o_ref):
            j = _ix(ix)
            o = jnp.dot(xbuf[:, pl.ds(0, 2048)], w_ref[...],
                        preferred_element_type=jnp.float32)
            o_ref[...] = o + db4[:, pl.ds(j * nb4, nb4)]

        pltpu.emit_pipeline(
            d4_body, grid=(out_cols // nb4,),
            in_specs=[pl.BlockSpec((2048, nb4), lambda j: (0, j))],
            out_specs=[pl.BlockSpec((1024, nb4), lambda j: (0, j))],
            _explicit_indices=True,
        )(dw4_hbm, out_hbm)

    xt_cp.wait()
    modality(0, 4096, w1t_hbm, w2t_hbm, dw3t_hbm, dw4t_hbm, w3bt, dw2bt,
             w3t_cp, dw2t_cp,
             (b1t_ref, b2t_ref, b3t_ref, w4t_ref, b4t_ref),
             (dw1t_ref, db1t_ref, db2t_ref, db3t_ref, db4t_ref),
             cbt_ref, tout_hbm, 4096, tloss_ref, tidx_ref, tzq_ref,
             lambda: xi_cp.start(),
             lambda: (w3i_cp.start(), dw2i_cp.start()))

    xi_cp.wait()
    modality(2048, 768, w1i_hbm, w2i_hbm, dw3i_hbm, dw4i_hbm, w3bt, dw2bt,
             w3i_cp, dw2i_cp,
             (b1i_ref, b2i_ref, b3i_ref, w4i_ref, b4i_ref),
             (dw1i_ref, db1i_ref, db2i_ref, db3i_ref, db4i_ref),
             cbi_ref, iout_hbm, 768, iloss_ref, iidx_ref, izq_ref,
             lambda: None, lambda: None)


def _mega(text_x, image_x, t_enc, t_dec, i_enc, i_dec, t_cb, i_cb):
    B = text_x.shape[0]
    L, K, E = t_cb.shape
    f32 = jnp.float32
    any_spec = pl.BlockSpec(memory_space=pl.ANY)
    vmem_spec = pl.BlockSpec(memory_space=pltpu.MemorySpace.VMEM)
    (W1t, b1t), (W2t, b2t), (W3t, b3t), (W4t, b4t) = t_enc
    (dW1t, db1t), (dW2t, db2t), (dW3t, db3t), (dW4t, db4t) = t_dec
    (W1i, b1i), (W2i, b2i), (W3i, b3i), (W4i, b4i) = i_enc
    (dW1i, db1i), (dW2i, db2i), (dW3i, db3i), (dW4i, db4i) = i_dec
    r2 = lambda b: b.reshape(1, -1)
    outs = pl.pallas_call(
        _mega_kernel,
        in_specs=(
            [any_spec, any_spec]
            + [any_spec, any_spec, any_spec, vmem_spec,
               vmem_spec, any_spec, any_spec, any_spec]
            + [any_spec, any_spec, any_spec, vmem_spec,
               vmem_spec, any_spec, any_spec, any_spec]
            + [vmem_spec] * 16
            + [vmem_spec, vmem_spec]
        ),
        out_specs=(any_spec, any_spec, vmem_spec, vmem_spec,
                   vmem_spec, vmem_spec, vmem_spec, vmem_spec),
        out_shape=(
            jax.ShapeDtypeStruct((B, 4096), f32),
            jax.ShapeDtypeStruct((B, 768), f32),
            jax.ShapeDtypeStruct((1, 1), f32),
            jax.ShapeDtypeStruct((1, 1), f32),
            jax.ShapeDtypeStruct((L, B), jnp.int32),
            jax.ShapeDtypeStruct((L, B), jnp.int32),
            jax.ShapeDtypeStruct((B, E), f32),
            jax.ShapeDtypeStruct((B, E), f32),
        ),
        scratch_shapes=[
            pltpu.VMEM((1024, 4096), f32),
            pltpu.VMEM((1024, 2048), f32),
            pltpu.VMEM((1024, 1024), f32),
            pltpu.VMEM((1024, 512), f32),
            pltpu.VMEM((512, 1024), f32),
            pltpu.SemaphoreType.DMA((8,)),
        ],
    )(text_x, image_x,
      W1t, W2t, W3t, W4t, dW1t, dW2t, dW3t, dW4t,
      W1i, W2i, W3i, W4i, dW1i, dW2i, dW3i, dW4i,
      r2(b1t), r2(b2t), r2(b3t), r2(b4t),
      r2(db1t), r2(db2t), r2(db3t), r2(db4t),
      r2(b1i), r2(b2i), r2(b3i), r2(b4i),
      r2(db1i), r2(db2i), r2(db3i), r2(db4i),
      t_cb, i_cb)
    (tout, iout, tls, ils, tidxT, iidxT, tzq, izq) = outs
    t_loss = 1.25 * tls[0, 0] / (B * E)
    i_loss = 1.25 * ils[0, 0] / (B * E)
    return (tout, iout, t_loss, i_loss, tidxT.T, iidxT.T, tzq, izq)


def kernel(text_x, image_x, t_enc, t_dec, i_enc, i_dec, t_cb, i_cb):
    return _mega(text_x, image_x, t_enc, t_dec, i_enc, i_dec, t_cb, i_cb)
